# baseline (speedup 1.0000x reference)
import functools

import jax
import jax.numpy as jnp
from jax import lax
from jax.experimental import pallas as pl
from jax.experimental.pallas import tpu as pltpu
from jax.experimental.pallas import tpu_sc as plsc

N_CAT = 100000
EMBED_DIM = 64
BATCH = 16384
FIELDS = 100

_INFO = plsc.get_sparse_core_info()
NC, NS = _INFO.num_cores, _INFO.num_subcores
NW = NC * NS

TOTAL = BATCH * FIELDS
B_PER_W = BATCH // NW
GB = 4
GROUP = GB * FIELDS
GROUPS = B_PER_W // GB
PAIRS = GROUPS // 2


def _body(obs_hbm, table_hbm, out_hbm,
          idx0, idx1, rows0, rows1, gsem, isem, wsem0, wsem1):
  wid = lax.axis_index("s") * NC + lax.axis_index("c")
  batch_base = wid * B_PER_W
  idx_bufs = (idx0, idx1)
  rows_bufs = (rows0, rows1)
  wsems = (wsem0, wsem1)

  def idx_src(g):
    return obs_hbm.at[pl.ds((batch_base + g * GB) * FIELDS, GROUP)]

  pltpu.async_copy(idx_src(0), idx0, isem)

  def pair(p, carry):
    for b in (0, 1):
      g = 2 * p + b
      idx_v, rows_v, wsem = idx_bufs[b], rows_bufs[b], wsems[b]
      @pl.when(p > 0)
      def _():
        for i in range(GB):
          pltpu.make_async_copy(
              rows_v.at[pl.ds(i * FIELDS, FIELDS)],
              out_hbm.at[batch_base], wsem).wait()
      pltpu.make_async_copy(idx_src(g), idx_v, isem).wait()
      gather = pltpu.async_copy(table_hbm.at[idx_v], rows_v, gsem)
      @pl.when(g + 1 < GROUPS)
      def _():
        pltpu.async_copy(idx_src(g + 1), idx_bufs[1 - b], isem)
      gather.wait()
      for i in range(GB):
        pltpu.async_copy(
            rows_v.at[pl.ds(i * FIELDS, FIELDS)],
            out_hbm.at[batch_base + g * GB + i],
            wsem,
        )
    return carry

  lax.fori_loop(0, PAIRS, pair, 0)

  for b in (0, 1):
    for i in range(GB):
      pltpu.make_async_copy(
          rows_bufs[b].at[pl.ds(i * FIELDS, FIELDS)],
          out_hbm.at[batch_base], wsems[b]).wait()


@jax.jit
def kernel(obs, table):
  idx = obs.reshape(TOTAL).astype(jnp.int32)
  mesh = plsc.VectorSubcoreMesh(core_axis_name="c", subcore_axis_name="s")
  return pl.kernel(
      _body,
      out_type=jax.ShapeDtypeStruct((BATCH, FIELDS, EMBED_DIM), jnp.float32),
      mesh=mesh,
      scratch_types=[
          pltpu.VMEM((GROUP,), jnp.int32),
          pltpu.VMEM((GROUP,), jnp.int32),
          pltpu.VMEM((GROUP, EMBED_DIM), jnp.float32),
          pltpu.VMEM((GROUP, EMBED_DIM), jnp.float32),
          pltpu.SemaphoreType.DMA,
          pltpu.SemaphoreType.DMA,
          pltpu.SemaphoreType.DMA,
          pltpu.SemaphoreType.DMA,
      ],
      compiler_params=pltpu.CompilerParams(use_tc_tiling_on_sc=False),
  )(idx, table)
